# Initial kernel scaffold; baseline (speedup 1.0000x reference)
import functools

import jax
import jax.numpy as jnp
from jax import lax
from jax.experimental import pallas as pl
from jax.experimental.pallas import tpu as pltpu
from jax.experimental.pallas import tpu_sc as plsc

_NUM_VOCAB = 100000
_MAXLEN = 200
_HID = 64
_BATCH = 4096
_SEQ = 200

_NC = 2
_NS = 16
_NW = _NC * _NS
_TOTAL = _BATCH * _SEQ
_PER_W = _TOTAL // _NW
_STEPS = _PER_W // _SEQ
_SUB = 100
_NSUB = _SEQ // _SUB


def _sc_embed(x_flat, emb_weight, pos_emb_weight):
  mesh = plsc.VectorSubcoreMesh(core_axis_name="c", subcore_axis_name="s")

  @functools.partial(
      pl.kernel,
      out_type=jax.ShapeDtypeStruct((_TOTAL, _HID), jnp.float32),
      mesh=mesh,
      scratch_types=[
          pltpu.VMEM((_PER_W,), jnp.int32),
          pltpu.VMEM((_SEQ, _HID), jnp.float32),
          pltpu.VMEM((_SEQ, _HID), jnp.float32),
          pltpu.SemaphoreType.DMA,
      ],
  )
  def k(x_hbm, emb_hbm, pos_hbm, out_hbm, idx_v, rows_v, pos_v, sem):
    wid = lax.axis_index("s") * _NC + lax.axis_index("c")
    wbase = wid * _PER_W
    pltpu.sync_copy(pos_hbm, pos_v)
    pltpu.sync_copy(x_hbm.at[pl.ds(wbase, _PER_W)], idx_v)

    def step(r, carry):
      row0 = r * _SEQ
      copies = []
      for c in range(_NSUB):
        copies.append(
            pltpu.async_copy(
                emb_hbm.at[idx_v.at[pl.ds(row0 + c * _SUB, _SUB)]],
                rows_v.at[pl.ds(c * _SUB, _SUB)],
                sem,
            ))
      for cp in copies:
        cp.wait()

      def add_row(s, carry2):
        for c in range(_HID // 16):
          sl = pl.ds(c * 16, 16)
          rows_v[s, sl] = rows_v[s, sl] + pos_v[s, sl]
        return carry2

      lax.fori_loop(0, _SEQ, add_row, 0)
      pltpu.sync_copy(rows_v, out_hbm.at[pl.ds(wbase + row0, _SEQ)])
      return carry

    lax.fori_loop(0, _STEPS, step, 0)

  return k(x_flat, emb_weight, pos_emb_weight)


def kernel(x, emb_weight, pos_emb_weight):
  x_flat = x.reshape(-1).astype(jnp.int32)
  out = _sc_embed(x_flat, emb_weight, pos_emb_weight)
  return out.reshape(_BATCH, _SEQ, _HID)


# speedup vs baseline: 3.2920x; 3.2920x over previous
import functools

import jax
import jax.numpy as jnp
from jax import lax
from jax.experimental import pallas as pl
from jax.experimental.pallas import tpu as pltpu
from jax.experimental.pallas import tpu_sc as plsc

_NUM_VOCAB = 100000
_MAXLEN = 200
_HID = 64
_BATCH = 4096
_SEQ = 200

_NC = 2
_NS = 16
_NW = _NC * _NS
_TOTAL = _BATCH * _SEQ
_PER_W = _TOTAL // _NW
_STEPS = _PER_W // _SEQ
_SUBS = ((0, 128), (128, 72))


def _sc_embed(x_flat, emb_weight, pos_emb_weight):
  mesh = plsc.VectorSubcoreMesh(core_axis_name="c", subcore_axis_name="s")

  @functools.partial(
      pl.kernel,
      out_type=jax.ShapeDtypeStruct((_TOTAL, _HID), jnp.float32),
      mesh=mesh,
      compiler_params=pltpu.CompilerParams(use_tc_tiling_on_sc=False),
      scratch_types=[
          pltpu.VMEM((_PER_W,), jnp.int32),
          pltpu.VMEM((_SEQ, _HID), jnp.float32),
          pltpu.VMEM((_SEQ, _HID), jnp.float32),
          pltpu.SemaphoreType.DMA,
      ],
  )
  def k(x_hbm, emb_hbm, pos_hbm, out_hbm, idx_v, rows_v, pos_v, sem):
    wid = lax.axis_index("s") * _NC + lax.axis_index("c")
    wbase = wid * _PER_W
    pltpu.sync_copy(pos_hbm, pos_v)
    pltpu.sync_copy(x_hbm.at[pl.ds(wbase, _PER_W)], idx_v)

    def step(r, carry):
      row0 = r * _SEQ
      copies = []
      for off, n in _SUBS:
        copies.append(
            pltpu.async_copy(
                emb_hbm.at[idx_v.at[pl.ds(row0 + off, n)]],
                rows_v.at[pl.ds(off, n)],
                sem,
            ))
      for cp in copies:
        cp.wait()

      def add_row(s, carry2):
        for c in range(_HID // 16):
          sl = pl.ds(c * 16, 16)
          rows_v[s, sl] = rows_v[s, sl] + pos_v[s, sl]
        return carry2

      lax.fori_loop(0, _SEQ, add_row, 0)
      pltpu.sync_copy(rows_v, out_hbm.at[pl.ds(wbase + row0, _SEQ)])
      return carry

    lax.fori_loop(0, _STEPS, step, 0)

  return k(x_flat, emb_weight, pos_emb_weight)


def kernel(x, emb_weight, pos_emb_weight):
  x_flat = x.reshape(-1).astype(jnp.int32)
  out = _sc_embed(x_flat, emb_weight, pos_emb_weight)
  return out.reshape(_BATCH, _SEQ, _HID)
